# baseline (speedup 1.0000x reference)
import functools

import jax
import jax.numpy as jnp
from jax import lax
from jax.experimental import pallas as pl
from jax.experimental.pallas import tpu as pltpu
from jax.experimental.pallas import tpu_sc as plsc

_NC = 2
_NS = 16
_NW = _NC * _NS


def _sc_call(xf, emb_table, B, C, F, T):
    G = B * C
    g_per_w = G // _NW

    mesh = plsc.VectorSubcoreMesh(core_axis_name="c", subcore_axis_name="s")

    @functools.partial(
        pl.kernel,
        mesh=mesh,
        out_type=jax.ShapeDtypeStruct((G, F, T), jnp.float32),
        scratch_types=[
            pltpu.VMEM((256 * 128,), jnp.float32),
            pltpu.VMEM((F, T), jnp.float32),
        ],
    )
    def k(x_hbm, emb_hbm, out_hbm, emb_v, buf):
        wid = lax.axis_index("s") * _NC + lax.axis_index("c")
        pltpu.sync_copy(emb_hbm, emb_v)
        g0 = wid * g_per_w

        for r in range(16):

            def group_body(j, carry, r=r):
                g = g0 + j * 16 + r
                c_idx = lax.rem(g, C)
                cb = (c_idx // 16) * 16
                pltpu.sync_copy(x_hbm.at[g], buf)

                def row_body(f, carry2):
                    v = emb_v[pl.ds(f * C + cb, 16)]
                    fe = jnp.full((16,), v[r], jnp.float32)
                    for tc in range(T // 16):
                        sl = pl.ds(tc * 16, 16)
                        buf[f, sl] = buf[f, sl] + fe
                    return carry2

                lax.fori_loop(0, F, row_body, 0)
                pltpu.sync_copy(buf, out_hbm.at[g])
                return carry

            lax.fori_loop(0, g_per_w // 16, group_body, 0)

    return k(xf, emb_table)


def kernel(x, emb_table):
    b, c, f, t = x.shape
    out = _sc_call(x.reshape(b * c, f, t), emb_table.reshape(-1), b, c, f, t)
    return out.reshape(b, c, f, t)
